# speedup vs baseline: 1.6748x; 1.6748x over previous
import functools

import jax
import jax.numpy as jnp
from jax import lax
from jax.experimental import pallas as pl
from jax.experimental.pallas import tpu as pltpu
from jax.experimental.pallas import tpu_sc as plsc

VOCAB = 256000
EMBED_DIM = 768
BATCH = 4096
SEQ = 64

NUM_ROWS = BATCH * SEQ
NUM_CORES = 2
NUM_SUBCORES = 16
NUM_WORKERS = NUM_CORES * NUM_SUBCORES
ROWS_PER_WORKER = NUM_ROWS // NUM_WORKERS
CHUNK = 64
NUM_CHUNKS = ROWS_PER_WORKER // CHUNK


def _make_kernel():
    mesh = plsc.VectorSubcoreMesh(
        core_axis_name="c", subcore_axis_name="s",
        num_cores=NUM_CORES, num_subcores=NUM_SUBCORES)

    @functools.partial(
        pl.kernel,
        mesh=mesh,
        out_type=jax.ShapeDtypeStruct((NUM_ROWS, EMBED_DIM), jnp.float32),
        scratch_types=[
            pltpu.VMEM((NUM_CHUNKS, CHUNK), jnp.int32),
            pltpu.VMEM((CHUNK, EMBED_DIM), jnp.float32),
            pltpu.SemaphoreType.DMA,
        ],
    )
    def gather_kernel(idx_hbm, table_hbm, out_hbm, idx_v, rows_v, sem):
        wid = lax.axis_index("s") * NUM_CORES + lax.axis_index("c")
        base = wid * ROWS_PER_WORKER
        pltpu.sync_copy(idx_hbm.at[wid], idx_v)

        def body(i, carry):
            pltpu.async_copy(table_hbm.at[idx_v.at[i]], rows_v, sem).wait()
            pltpu.sync_copy(rows_v, out_hbm.at[pl.ds(base + i * CHUNK, CHUNK)])
            return carry

        lax.fori_loop(0, NUM_CHUNKS, body, 0)

    return gather_kernel


_gather = _make_kernel()


def kernel(text, table):
    idx = text.reshape(NUM_WORKERS, NUM_CHUNKS, CHUNK).astype(jnp.int32)
    out = _gather(idx, table)
    return out.reshape(BATCH, SEQ, EMBED_DIM)


# speedup vs baseline: 1.8798x; 1.1224x over previous
import functools

import jax
import jax.numpy as jnp
from jax import lax
from jax.experimental import pallas as pl
from jax.experimental.pallas import tpu as pltpu
from jax.experimental.pallas import tpu_sc as plsc

VOCAB = 256000
EMBED_DIM = 768
BATCH = 4096
SEQ = 64

NUM_ROWS = BATCH * SEQ
NUM_CORES = 2
NUM_SUBCORES = 16
NUM_WORKERS = NUM_CORES * NUM_SUBCORES
ROWS_PER_WORKER = NUM_ROWS // NUM_WORKERS
CHUNK = 32
NUM_CHUNKS = ROWS_PER_WORKER // CHUNK
NBUF = 4


def _make_kernel():
    mesh = plsc.VectorSubcoreMesh(
        core_axis_name="c", subcore_axis_name="s",
        num_cores=NUM_CORES, num_subcores=NUM_SUBCORES)

    @functools.partial(
        pl.kernel,
        mesh=mesh,
        out_type=jax.ShapeDtypeStruct((NUM_ROWS, EMBED_DIM), jnp.float32),
        scratch_types=[
            pltpu.VMEM((NUM_CHUNKS, CHUNK), jnp.int32),
            pltpu.VMEM((NBUF, CHUNK, EMBED_DIM), jnp.float32),
            pltpu.SemaphoreType.DMA,
            pltpu.SemaphoreType.DMA,
        ],
    )
    def gather_kernel(idx_hbm, table_hbm, out_hbm, idx_v, rows_v, sem_in,
                      sem_out):
        wid = lax.axis_index("s") * NUM_CORES + lax.axis_index("c")
        base = wid * ROWS_PER_WORKER
        pltpu.sync_copy(idx_hbm.at[wid], idx_v)

        for b in range(NBUF):
            pltpu.async_copy(table_hbm.at[idx_v.at[b]], rows_v.at[b], sem_in)

        def body(g, carry):
            for b in range(NBUF):
                i = g * NBUF + b
                prev_b = (b - 1) % NBUF
                @pl.when(jnp.logical_and(i >= 1, i + NBUF - 1 < NUM_CHUNKS))
                def _():
                    pltpu.make_async_copy(
                        rows_v.at[prev_b],
                        out_hbm.at[pl.ds(base + (i - 1) * CHUNK, CHUNK)],
                        sem_out).wait()
                    pltpu.async_copy(
                        table_hbm.at[idx_v.at[i + NBUF - 1]],
                        rows_v.at[prev_b], sem_in)
                pltpu.make_async_copy(
                    table_hbm.at[idx_v.at[i]], rows_v.at[b], sem_in).wait()
                pltpu.async_copy(
                    rows_v.at[b],
                    out_hbm.at[pl.ds(base + i * CHUNK, CHUNK)], sem_out)
            return carry

        lax.fori_loop(0, NUM_CHUNKS // NBUF, body, 0)

        for b in range(NBUF):
            i = NUM_CHUNKS - NBUF + b
            pltpu.make_async_copy(
                rows_v.at[b],
                out_hbm.at[pl.ds(base + i * CHUNK, CHUNK)], sem_out).wait()

    return gather_kernel


_gather = _make_kernel()


def kernel(text, table):
    idx = text.reshape(NUM_WORKERS, NUM_CHUNKS, CHUNK).astype(jnp.int32)
    out = _gather(idx, table)
    return out.reshape(BATCH, SEQ, EMBED_DIM)
